# baseline (device time: 89877 ns/iter reference)
import jax
import jax.numpy as jnp
from jax import lax
from jax.experimental import pallas as pl
from jax.experimental.pallas import tpu as pltpu

N_DEV = 4
M_PER = 2048
CH = M_PER // N_DEV
K = 1024
N = 1024


def kernel(t, W):
    def body(t_ref, w_ref, out_ref, work_ref, rs_recv_ref,
             rs_send_sems, rs_recv_sems, ag_send_sems, ag_recv_sems):
        p = lax.axis_index("i")
        left = (p + N_DEV - 1) % N_DEV
        right = (p + 1) % N_DEV

        barrier_sem = pltpu.get_barrier_semaphore()
        for nbr in (left, right):
            pl.semaphore_signal(
                barrier_sem, inc=1,
                device_id=(nbr,), device_id_type=pl.DeviceIdType.MESH,
            )
        pl.semaphore_wait(barrier_sem, 2)

        for c in range(N_DEV):
            work_ref[c, :, :] = t_ref[pl.ds(c * CH, CH), :].astype(jnp.bfloat16)

        for s in range(N_DEV - 1):
            send_idx = (p - s + N_DEV) % N_DEV
            recv_idx = (p - s - 1 + N_DEV) % N_DEV
            rdma = pltpu.make_async_remote_copy(
                src_ref=work_ref.at[send_idx],
                dst_ref=rs_recv_ref.at[s],
                send_sem=rs_send_sems.at[s],
                recv_sem=rs_recv_sems.at[s],
                device_id=(right,),
                device_id_type=pl.DeviceIdType.MESH,
            )
            rdma.start()
            rdma.wait()
            work_ref[recv_idx] = work_ref[recv_idx] + rs_recv_ref[s]

        own = (p + 1) % N_DEV
        w_bf = w_ref[...].astype(jnp.bfloat16)
        acc = lax.dot_general(
            work_ref[own], w_bf,
            dimension_numbers=(((1,), (0,)), ((), ())),
            preferred_element_type=jnp.float32,
        )
        out_ref[pl.ds(own * CH, CH), :] = acc.astype(jnp.bfloat16)

        for h in range(N_DEV - 1):
            o = (p + 1 - h + N_DEV) % N_DEV
            rdma = pltpu.make_async_remote_copy(
                src_ref=out_ref.at[pl.ds(o * CH, CH), :],
                dst_ref=out_ref.at[pl.ds(o * CH, CH), :],
                send_sem=ag_send_sems.at[h],
                recv_sem=ag_recv_sems.at[h],
                device_id=(right,),
                device_id_type=pl.DeviceIdType.MESH,
            )
            rdma.start()
            rdma.wait()

    return pl.pallas_call(
        body,
        out_shape=jax.ShapeDtypeStruct((M_PER, N), jnp.bfloat16),
        in_specs=[
            pl.BlockSpec(memory_space=pltpu.VMEM),
            pl.BlockSpec(memory_space=pltpu.VMEM),
        ],
        out_specs=pl.BlockSpec(memory_space=pltpu.VMEM),
        scratch_shapes=[
            pltpu.VMEM((N_DEV, CH, K), jnp.bfloat16),
            pltpu.VMEM((N_DEV - 1, CH, K), jnp.bfloat16),
            pltpu.SemaphoreType.DMA((N_DEV - 1,)),
            pltpu.SemaphoreType.DMA((N_DEV - 1,)),
            pltpu.SemaphoreType.DMA((N_DEV - 1,)),
            pltpu.SemaphoreType.DMA((N_DEV - 1,)),
        ],
        compiler_params=pltpu.CompilerParams(collective_id=0),
    )(t, W)


# device time: 56429 ns/iter; 1.5927x vs baseline; 1.5927x over previous
import jax
import jax.numpy as jnp
from jax import lax
from jax.experimental import pallas as pl
from jax.experimental.pallas import tpu as pltpu

N_DEV = 4
M_PER = 2048
CH = M_PER // N_DEV
HH = CH // 2
K = 1024
N = 1024


def kernel(t, W):
    def body(t_ref, w_ref, out_ref, top_ref, bot_ref, rs_recv_ref,
             rs_send_sems, rs_recv_sems, ag_send_sems, ag_recv_sems):
        p = lax.axis_index("i")
        left = (p + N_DEV - 1) % N_DEV
        right = (p + 1) % N_DEV

        barrier_sem = pltpu.get_barrier_semaphore()
        for nbr in (left, right):
            pl.semaphore_signal(
                barrier_sem, inc=1,
                device_id=(nbr,), device_id_type=pl.DeviceIdType.MESH,
            )
        pl.semaphore_wait(barrier_sem, 2)

        for c in range(N_DEV):
            top_ref[c, :, :] = t_ref[pl.ds(c * CH, HH), :].astype(jnp.bfloat16)
            bot_ref[c, :, :] = t_ref[pl.ds(c * CH + HH, HH), :].astype(
                jnp.bfloat16)

        for s in range(N_DEV - 1):
            f_send = (p - s + N_DEV) % N_DEV
            f_recv = (p - s - 1 + N_DEV) % N_DEV
            b_send = (p + s) % N_DEV
            b_recv = (p + s + 1) % N_DEV
            fwd = pltpu.make_async_remote_copy(
                src_ref=top_ref.at[f_send],
                dst_ref=rs_recv_ref.at[0, s],
                send_sem=rs_send_sems.at[0, s],
                recv_sem=rs_recv_sems.at[0, s],
                device_id=(right,),
                device_id_type=pl.DeviceIdType.MESH,
            )
            bwd = pltpu.make_async_remote_copy(
                src_ref=bot_ref.at[b_send],
                dst_ref=rs_recv_ref.at[1, s],
                send_sem=rs_send_sems.at[1, s],
                recv_sem=rs_recv_sems.at[1, s],
                device_id=(left,),
                device_id_type=pl.DeviceIdType.MESH,
            )
            fwd.start()
            bwd.start()
            fwd.wait()
            bwd.wait()
            top_ref[f_recv] = top_ref[f_recv] + rs_recv_ref[0, s]
            bot_ref[b_recv] = bot_ref[b_recv] + rs_recv_ref[1, s]

        own_t = (p + 1) % N_DEV
        own_b = (p - 1 + N_DEV) % N_DEV
        w_bf = w_ref[...].astype(jnp.bfloat16)
        acc_t = lax.dot_general(
            top_ref[own_t], w_bf,
            dimension_numbers=(((1,), (0,)), ((), ())),
            preferred_element_type=jnp.float32,
        )
        out_ref[pl.ds(own_t * CH, HH), :] = acc_t.astype(jnp.bfloat16)
        acc_b = lax.dot_general(
            bot_ref[own_b], w_bf,
            dimension_numbers=(((1,), (0,)), ((), ())),
            preferred_element_type=jnp.float32,
        )
        out_ref[pl.ds(own_b * CH + HH, HH), :] = acc_b.astype(jnp.bfloat16)

        for h in range(N_DEV - 1):
            o_t = (p + 1 - h + N_DEV) % N_DEV
            o_b = (p - 1 + h + N_DEV) % N_DEV
            fwd = pltpu.make_async_remote_copy(
                src_ref=out_ref.at[pl.ds(o_t * CH, HH), :],
                dst_ref=out_ref.at[pl.ds(o_t * CH, HH), :],
                send_sem=ag_send_sems.at[0, h],
                recv_sem=ag_recv_sems.at[0, h],
                device_id=(right,),
                device_id_type=pl.DeviceIdType.MESH,
            )
            bwd = pltpu.make_async_remote_copy(
                src_ref=out_ref.at[pl.ds(o_b * CH + HH, HH), :],
                dst_ref=out_ref.at[pl.ds(o_b * CH + HH, HH), :],
                send_sem=ag_send_sems.at[1, h],
                recv_sem=ag_recv_sems.at[1, h],
                device_id=(left,),
                device_id_type=pl.DeviceIdType.MESH,
            )
            fwd.start()
            bwd.start()
            fwd.wait()
            bwd.wait()

    return pl.pallas_call(
        body,
        out_shape=jax.ShapeDtypeStruct((M_PER, N), jnp.bfloat16),
        in_specs=[
            pl.BlockSpec(memory_space=pltpu.VMEM),
            pl.BlockSpec(memory_space=pltpu.VMEM),
        ],
        out_specs=pl.BlockSpec(memory_space=pltpu.VMEM),
        scratch_shapes=[
            pltpu.VMEM((N_DEV, HH, K), jnp.bfloat16),
            pltpu.VMEM((N_DEV, HH, K), jnp.bfloat16),
            pltpu.VMEM((2, N_DEV - 1, HH, K), jnp.bfloat16),
            pltpu.SemaphoreType.DMA((2, N_DEV - 1)),
            pltpu.SemaphoreType.DMA((2, N_DEV - 1)),
            pltpu.SemaphoreType.DMA((2, N_DEV - 1)),
            pltpu.SemaphoreType.DMA((2, N_DEV - 1)),
        ],
        compiler_params=pltpu.CompilerParams(collective_id=0),
    )(t, W)


# device time: 45814 ns/iter; 1.9618x vs baseline; 1.2317x over previous
import jax
import jax.numpy as jnp
from jax import lax
from jax.experimental import pallas as pl
from jax.experimental.pallas import tpu as pltpu

N_DEV = 4
M_PER = 2048
CH = M_PER // N_DEV
HH = CH // 2
S = 2
SH = HH // S
K = 1024
N = 1024
FWD, BWD = 0, 1


def kernel(t, W):
    def body(t_ref, w_ref, out_ref, top_ref, bot_ref, w_bf_ref, rs_recv_ref,
             rs_send_sems, rs_recv_sems, ag_send_sems, ag_recv_sems):
        p = lax.axis_index("i")
        left = (p + N_DEV - 1) % N_DEV
        right = (p + 1) % N_DEV

        barrier_sem = pltpu.get_barrier_semaphore()
        for nbr in (left, right):
            pl.semaphore_signal(
                barrier_sem, inc=1,
                device_id=(nbr,), device_id_type=pl.DeviceIdType.MESH,
            )
        pl.semaphore_wait(barrier_sem, 2)

        def stage(c):
            for j in range(S):
                top_ref[c, j] = t_ref[
                    pl.ds(c * CH + j * SH, SH), :].astype(jnp.bfloat16)
                bot_ref[c, j] = t_ref[
                    pl.ds(c * CH + HH + j * SH, SH), :].astype(jnp.bfloat16)

        def rs_send_chunk(d, s):
            return (p - s + N_DEV) % N_DEV if d == FWD else (p + s) % N_DEV

        def rs_recv_chunk(d, s):
            return (p - s - 1 + N_DEV) % N_DEV if d == FWD \
                else (p + s + 1) % N_DEV

        def rs_rdma(d, s, j):
            buf = top_ref if d == FWD else bot_ref
            return pltpu.make_async_remote_copy(
                src_ref=buf.at[rs_send_chunk(d, s), j],
                dst_ref=rs_recv_ref.at[d, s, j],
                send_sem=rs_send_sems.at[d, s, j],
                recv_sem=rs_recv_sems.at[d, s, j],
                device_id=(right if d == FWD else left,),
                device_id_type=pl.DeviceIdType.MESH,
            )

        def ag_rows(d, h):
            if d == FWD:
                o = (p + 1 - h + N_DEV) % N_DEV
                return o * CH
            o = (p - 1 + h + N_DEV) % N_DEV
            return o * CH + HH

        def ag_rdma(d, h, j):
            rows = ag_rows(d, h)
            sl = out_ref.at[pl.ds(rows + j * SH, SH), :]
            return pltpu.make_async_remote_copy(
                src_ref=sl, dst_ref=sl,
                send_sem=ag_send_sems.at[d, h, j],
                recv_sem=ag_recv_sems.at[d, h, j],
                device_id=(right if d == FWD else left,),
                device_id_type=pl.DeviceIdType.MESH,
            )

        stage_first = rs_send_chunk(FWD, 0)
        del stage_first
        for c in range(N_DEV):
            stage(c)
        for d in (FWD, BWD):
            for j in range(S):
                rs_rdma(d, 0, j).start()
        w_bf_ref[...] = w_ref[...].astype(jnp.bfloat16)

        own = {FWD: (p + 1) % N_DEV, BWD: (p - 1 + N_DEV) % N_DEV}

        for s in range(N_DEV - 1):
            for j in range(S):
                for d in (FWD, BWD):
                    buf = top_ref if d == FWD else bot_ref
                    rs_rdma(d, s, j).wait_recv()
                    rc = rs_recv_chunk(d, s)
                    buf[rc, j] = buf[rc, j] + rs_recv_ref[d, s, j]
                    if s < N_DEV - 2:
                        rs_rdma(d, s + 1, j).start()
                    else:
                        acc = lax.dot_general(
                            buf[own[d], j], w_bf_ref[...],
                            dimension_numbers=(((1,), (0,)), ((), ())),
                            preferred_element_type=jnp.float32,
                        )
                        out_ref[pl.ds(ag_rows(d, 0) + j * SH, SH), :] = (
                            acc.astype(jnp.bfloat16))
                        ag_rdma(d, 0, j).start()

        for h in range(N_DEV - 1):
            for j in range(S):
                for d in (FWD, BWD):
                    ag_rdma(d, h, j).wait_recv()
                    if h < N_DEV - 2:
                        ag_rdma(d, h + 1, j).start()

        for s in range(N_DEV - 1):
            for j in range(S):
                for d in (FWD, BWD):
                    rs_rdma(d, s, j).wait_send()
                    ag_rdma(d, s, j).wait_send()

    return pl.pallas_call(
        body,
        out_shape=jax.ShapeDtypeStruct((M_PER, N), jnp.bfloat16),
        in_specs=[
            pl.BlockSpec(memory_space=pltpu.VMEM),
            pl.BlockSpec(memory_space=pltpu.VMEM),
        ],
        out_specs=pl.BlockSpec(memory_space=pltpu.VMEM),
        scratch_shapes=[
            pltpu.VMEM((N_DEV, S, SH, K), jnp.bfloat16),
            pltpu.VMEM((N_DEV, S, SH, K), jnp.bfloat16),
            pltpu.VMEM((K, N), jnp.bfloat16),
            pltpu.VMEM((2, N_DEV - 1, S, SH, K), jnp.bfloat16),
            pltpu.SemaphoreType.DMA((2, N_DEV - 1, S)),
            pltpu.SemaphoreType.DMA((2, N_DEV - 1, S)),
            pltpu.SemaphoreType.DMA((2, N_DEV - 1, S)),
            pltpu.SemaphoreType.DMA((2, N_DEV - 1, S)),
        ],
        compiler_params=pltpu.CompilerParams(collective_id=0),
    )(t, W)
